# baseline (speedup 1.0000x reference)
import jax
import jax.numpy as jnp
from jax.experimental import pallas as pl
from jax.experimental.pallas import tpu as pltpu

_BINS = 10
_SCALE = 10.0 - 0.0001
_ROWS = 16384
_COLS = 1024
_BLOCK_ROWS = 256
_N_STEPS = _ROWS // _BLOCK_ROWS
_LOG2E = 1.4426950408889634
_LN2 = 0.6931471805599453


def _ghm_kernel(x_ref, t_ref, out_ref, cnt_ref, sum_ref):
    step = pl.program_id(0)

    @pl.when(step == 0)
    def _init():
        cnt_ref[...] = jnp.zeros_like(cnt_ref)
        sum_ref[...] = jnp.zeros_like(sum_ref)

    zeroh = jnp.zeros((16, 128), jnp.bfloat16)
    oneh = jnp.bfloat16(1.0)
    acc_c = [zeroh] * _BINS
    acc_s = [zeroh] * _BINS

    acc_all = zeroh

    for i in range(_BLOCK_ROWS // 16):
        for j in range(_COLS // 128):
            x = x_ref[pl.ds(16 * i, 16), pl.ds(128 * j, 128)]
            t = t_ref[pl.ds(16 * i, 16), pl.ds(128 * j, 128)]
            ax = jnp.abs(x)
            e = jnp.exp2(ax * (-_LOG2E))
            one_pe = e + 1.0
            d = 1.0 / one_pe
            t2 = jnp.where(x >= 0.0, t, 1.0 - t)
            g = jnp.abs(d - t2)
            binf = jnp.floor(g * _SCALE)
            bce = ax * (1.0 - t2) + jnp.log2(one_pe) * _LN2
            binh = binf.astype(jnp.bfloat16)
            bceh = bce.astype(jnp.bfloat16)
            for b in range(_BINS):
                mask = binh == jnp.bfloat16(b)
                if b < _BINS - 1:
                    acc_c[b] = acc_c[b] + jnp.where(mask, oneh, zeroh)
                acc_s[b] = acc_s[b] + jnp.where(mask, bceh, zeroh)

    for b in range(_BINS - 1):
        cnt_ref[pl.ds(16 * b, 16), :] += acc_c[b].astype(jnp.float32)
    for b in range(_BINS):
        sum_ref[pl.ds(16 * b, 16), :] += acc_s[b].astype(jnp.float32)

    @pl.when(step == _N_STEPS - 1)
    def _finish():
        counts = [jnp.sum(cnt_ref[pl.ds(16 * b, 16), :])
                  for b in range(_BINS - 1)]
        bsums = [jnp.sum(sum_ref[pl.ds(16 * b, 16), :])
                 for b in range(_BINS)]
        counts.append(jnp.float32(_ROWS * _COLS) - sum(counts))
        nonempty = jnp.float32(0.0)
        for b in range(_BINS):
            nonempty += jnp.where(counts[b] > 0.0, jnp.float32(1.0),
                                  jnp.float32(0.0))
        loss = jnp.float32(0.0)
        for b in range(_BINS):
            gd = jnp.maximum(counts[b] * nonempty, jnp.float32(0.0001))
            loss += bsums[b] / gd
        out_ref[...] = jnp.full((8, 128), loss, dtype=jnp.float32)


@jax.jit
def kernel(x, target):
    out = pl.pallas_call(
        _ghm_kernel,
        grid=(_N_STEPS,),
        in_specs=[
            pl.BlockSpec((_BLOCK_ROWS, _COLS), lambda i: (i, 0)),
            pl.BlockSpec((_BLOCK_ROWS, _COLS), lambda i: (i, 0)),
        ],
        out_specs=pl.BlockSpec((8, 128), lambda i: (0, 0)),
        out_shape=jax.ShapeDtypeStruct((8, 128), jnp.float32),
        scratch_shapes=[
            pltpu.VMEM((16 * _BINS, 128), jnp.float32),
            pltpu.VMEM((16 * _BINS, 128), jnp.float32),
        ],
    )(x, target)
    return out[0, 0]


# speedup vs baseline: 1.0035x; 1.0035x over previous
import jax
import jax.numpy as jnp
from jax.experimental import pallas as pl
from jax.experimental.pallas import tpu as pltpu

_BINS = 10
_SCALE = 10.0 - 0.0001
_ROWS = 16384
_COLS = 1024
_BLOCK_ROWS = 256
_N_STEPS = _ROWS // _BLOCK_ROWS
_LOG2E = 1.4426950408889634
_LN2 = 0.6931471805599453


def _ghm_kernel(x_ref, t_ref, out_ref, cnt_ref, sum_ref):
    step = pl.program_id(0)

    @pl.when(step == 0)
    def _init():
        cnt_ref[...] = jnp.zeros_like(cnt_ref)
        sum_ref[...] = jnp.zeros_like(sum_ref)

    zeroh = jnp.zeros((16, 128), jnp.bfloat16)
    oneh = jnp.bfloat16(1.0)
    acc_c = [zeroh] * _BINS
    acc_s = [zeroh] * _BINS

    for i in range(_BLOCK_ROWS // 16):
        for j in range(_COLS // 128):
            x = x_ref[pl.ds(16 * i, 16), pl.ds(128 * j, 128)]
            t = t_ref[pl.ds(16 * i, 16), pl.ds(128 * j, 128)]
            ax = jnp.abs(x)
            e = jnp.exp2(ax * (-_LOG2E))
            one_pe = e + 1.0
            d = 1.0 / one_pe
            sig = jnp.where(x >= 0.0, d, 1.0 - d)
            g = jnp.abs(sig - t)
            binf = jnp.floor(g * _SCALE)
            bce = jnp.maximum(x, 0.0) - x * t + jnp.log2(one_pe) * _LN2
            binh = binf.astype(jnp.bfloat16)
            bceh = bce.astype(jnp.bfloat16)
            for b in range(_BINS):
                mask = binh == jnp.bfloat16(b)
                if b < _BINS - 1:
                    acc_c[b] = acc_c[b] + jnp.where(mask, oneh, zeroh)
                acc_s[b] = acc_s[b] + jnp.where(mask, bceh, zeroh)

    for b in range(_BINS - 1):
        cnt_ref[pl.ds(16 * b, 16), :] += acc_c[b].astype(jnp.float32)
    for b in range(_BINS):
        sum_ref[pl.ds(16 * b, 16), :] += acc_s[b].astype(jnp.float32)

    @pl.when(step == _N_STEPS - 1)
    def _finish():
        counts = [jnp.sum(cnt_ref[pl.ds(16 * b, 16), :])
                  for b in range(_BINS - 1)]
        bsums = [jnp.sum(sum_ref[pl.ds(16 * b, 16), :]) for b in range(_BINS)]
        counts.append(jnp.float32(_ROWS * _COLS) - sum(counts))
        nonempty = jnp.float32(0.0)
        for b in range(_BINS):
            nonempty += jnp.where(counts[b] > 0.0, jnp.float32(1.0),
                                  jnp.float32(0.0))
        loss = jnp.float32(0.0)
        for b in range(_BINS):
            gd = jnp.maximum(counts[b] * nonempty, jnp.float32(0.0001))
            loss += bsums[b] / gd
        out_ref[...] = jnp.full((8, 128), loss, dtype=jnp.float32)


@jax.jit
def kernel(x, target):
    out = pl.pallas_call(
        _ghm_kernel,
        grid=(_N_STEPS,),
        in_specs=[
            pl.BlockSpec((_BLOCK_ROWS, _COLS), lambda i: (i, 0)),
            pl.BlockSpec((_BLOCK_ROWS, _COLS), lambda i: (i, 0)),
        ],
        out_specs=pl.BlockSpec((8, 128), lambda i: (0, 0)),
        out_shape=jax.ShapeDtypeStruct((8, 128), jnp.float32),
        scratch_shapes=[
            pltpu.VMEM((16 * _BINS, 128), jnp.float32),
            pltpu.VMEM((16 * _BINS, 128), jnp.float32),
        ],
    )(x, target)
    return out[0, 0]


# speedup vs baseline: 1.0123x; 1.0088x over previous
import jax
import jax.numpy as jnp
from jax.experimental import pallas as pl
from jax.experimental.pallas import tpu as pltpu

_BINS = 10
_SCALE = 10.0 - 0.0001
_ROWS = 16384
_COLS = 1024
_BLOCK_ROWS = 256
_N_STEPS = _ROWS // _BLOCK_ROWS
_LOG2E = 1.4426950408889634
_LN2 = 0.6931471805599453


def _ghm_kernel(x_ref, t_ref, out_ref, cnt_ref, sum_ref):
    step = pl.program_id(0)

    @pl.when(step == 0)
    def _init():
        cnt_ref[...] = jnp.zeros_like(cnt_ref)
        sum_ref[...] = jnp.zeros_like(sum_ref)

    zeroh = jnp.zeros((16, 128), jnp.bfloat16)
    oneh = jnp.bfloat16(1.0)
    acc_c = [zeroh] * _BINS
    acc_s = [zeroh] * _BINS

    for i in range(_BLOCK_ROWS // 16):
        for j in range(_COLS // 128):
            x = x_ref[pl.ds(16 * i, 16), pl.ds(128 * j, 128)]
            t = t_ref[pl.ds(16 * i, 16), pl.ds(128 * j, 128)]
            ax = jnp.abs(x)
            d = 0.5 * jnp.tanh(ax * 0.5) + 0.5
            sig = jnp.where(x >= 0.0, d, 1.0 - d)
            g = jnp.abs(sig - t)
            binf = jnp.floor(g * _SCALE)
            bce = jnp.maximum(x, 0.0) - x * t - jnp.log2(d) * _LN2
            binh = binf.astype(jnp.bfloat16)
            bceh = bce.astype(jnp.bfloat16)
            for b in range(_BINS):
                mask = binh == jnp.bfloat16(b)
                acc_c[b] = acc_c[b] + jnp.where(mask, oneh, zeroh)
                acc_s[b] = acc_s[b] + jnp.where(mask, bceh, zeroh)

    for b in range(_BINS):
        cnt_ref[pl.ds(16 * b, 16), :] += acc_c[b].astype(jnp.float32)
        sum_ref[pl.ds(16 * b, 16), :] += acc_s[b].astype(jnp.float32)

    @pl.when(step == _N_STEPS - 1)
    def _finish():
        counts = [jnp.sum(cnt_ref[pl.ds(16 * b, 16), :]) for b in range(_BINS)]
        bsums = [jnp.sum(sum_ref[pl.ds(16 * b, 16), :]) for b in range(_BINS)]
        nonempty = jnp.float32(0.0)
        for b in range(_BINS):
            nonempty += jnp.where(counts[b] > 0.0, jnp.float32(1.0),
                                  jnp.float32(0.0))
        loss = jnp.float32(0.0)
        for b in range(_BINS):
            gd = jnp.maximum(counts[b] * nonempty, jnp.float32(0.0001))
            loss += bsums[b] / gd
        out_ref[...] = jnp.full((8, 128), loss, dtype=jnp.float32)


@jax.jit
def kernel(x, target):
    out = pl.pallas_call(
        _ghm_kernel,
        grid=(_N_STEPS,),
        in_specs=[
            pl.BlockSpec((_BLOCK_ROWS, _COLS), lambda i: (i, 0)),
            pl.BlockSpec((_BLOCK_ROWS, _COLS), lambda i: (i, 0)),
        ],
        out_specs=pl.BlockSpec((8, 128), lambda i: (0, 0)),
        out_shape=jax.ShapeDtypeStruct((8, 128), jnp.float32),
        scratch_shapes=[
            pltpu.VMEM((16 * _BINS, 128), jnp.float32),
            pltpu.VMEM((16 * _BINS, 128), jnp.float32),
        ],
    )(x, target)
    return out[0, 0]
